# baseline (speedup 1.0000x reference)
import jax
import jax.numpy as jnp
from jax import lax
from jax.experimental import pallas as pl
from jax.experimental.pallas import tpu as pltpu
from jax.experimental.pallas import tpu_sc as plsc

_DSQ_THRESHOLD = 2.1025002
_R = 1024

_B = 8
_N = 2048
_D = 64
_TM = 512
_NC = 2
_NS = 16
_WPB = 4
_CHUNK = _N // _WPB
_OUT_N = _R + _N


def _match_body(mp_ref, rank_ref, nidx_ref, nmr_ref, nmc_ref):
    bn = mp_ref[:, pl.ds(_D, _D)]
    bn2 = bn + bn
    bsq = bn * bn
    onesd = jnp.ones((1, _D), jnp.float32)
    b2 = lax.dot_general(onesd, bsq, (((1,), (1,)), ((), ())),
                         preferred_element_type=jnp.float32)
    ones = jnp.ones((_N, 128), jnp.float32)
    neg_inf = jnp.float32(-jnp.inf)
    for t in range(_N // _TM):
        a_t = mp_ref[pl.ds(t * _TM, _TM), pl.ds(0, _D)]
        s = lax.dot_general(a_t, bn, (((1,), (1,)), ((), ())),
                            preferred_element_type=jnp.float32)
        s2 = lax.dot_general(a_t, bn2, (((1,), (1,)), ((), ())),
                             preferred_element_type=jnp.float32)
        a2t = jnp.sum(a_t * a_t, axis=1, keepdims=True)
        dsq = a2t + b2 - s2
        sm = jnp.where(dsq > _DSQ_THRESHOLD, neg_inf, s)
        colid128 = lax.broadcasted_iota(jnp.int32, (_TM, 128), 1)
        run_v = sm[:, 0:128]
        run_i = colid128
        for cix in range(1, _N // 128):
            v = sm[:, cix * 128:(cix + 1) * 128]
            upd = v > run_v
            run_v = jnp.where(upd, v, run_v)
            run_i = jnp.where(upd, colid128 + cix * 128, run_i)
        tmax = jnp.max(run_v, axis=1, keepdims=True)
        idx = jnp.min(jnp.where(run_v == tmax, run_i, _N), axis=1)
        nidx_ref[0, pl.ds(t * _TM, _TM)] = idx
        nmr_ref[0, pl.ds(t * _TM, _TM)] = jnp.max(run_v, axis=1)
        nmc_ref[pl.ds(t * _TM, _TM), :] = tmax
    vrow = nmr_ref[...]
    for t in range(_N // _TM):
        vi = nmc_ref[pl.ds(t * _TM, _TM), :]
        colid = lax.broadcasted_iota(jnp.int32, (_TM, _N), 1)
        rowid = lax.broadcasted_iota(jnp.int32, (_TM, _N), 0) + t * _TM
        cmp = (vrow > vi) | ((vrow == vi) & (colid < rowid))
        cmpf = jnp.where(cmp, 1.0, 0.0).astype(jnp.float32)
        cnt = lax.dot_general(cmpf, ones, (((1,), (0,)), ((), ())),
                              preferred_element_type=jnp.float32)
        rank_ref[0, pl.ds(t * _TM, _TM)] = cnt[:, 0].astype(jnp.int32)


def _match_call(mp, interpret=False):
    nb = mp.shape[0]
    return pl.pallas_call(
        _match_body,
        grid=(nb,),
        in_specs=[
            pl.BlockSpec((None, _N, 2 * _D), lambda b: (b, 0, 0)),
        ],
        out_specs=[
            pl.BlockSpec((None, 1, _N), lambda b: (b, 0, 0)),
            pl.BlockSpec((None, 1, _N), lambda b: (b, 0, 0)),
        ],
        out_shape=[
            jax.ShapeDtypeStruct((nb, 1, _N), jnp.int32),
            jax.ShapeDtypeStruct((nb, 1, _N), jnp.int32),
        ],
        scratch_shapes=[
            pltpu.VMEM((1, _N), jnp.float32),
            pltpu.VMEM((_N, 1), jnp.float32),
        ],
        interpret=interpret,
    )(mp)


_HB = 2
_BPW = _HB // _NC
_WPB2 = _NS // _BPW
_CH = _N // _WPB2


def _merge_body(h0, xp_hbm, rank_hbm, nidx_hbm, out_hbm,
                rank_v, nidx_v, tgt_v, rows_v, zbuf_v, image_sh):
    c = lax.axis_index("c")
    s = lax.axis_index("s")
    lb = s // _WPB2
    k = s % _WPB2
    base = k * _CH
    img0 = lb * _OUT_N
    b = c * _BPW + lb

    zero16 = jnp.zeros((16,), jnp.float32)
    for rr in range(16):
        for cc in range(2 * _D // 16):
            zbuf_v[rr, pl.ds(cc * 16, 16)] = zero16

    pltpu.sync_copy(rank_hbm.at[pl.ds(b * _N + base, _CH)], rank_v)
    pltpu.sync_copy(nidx_hbm.at[pl.ds(b * _N + base, _CH)], nidx_v)
    pltpu.sync_copy(xp_hbm.at[pl.ds((h0 + b) * _N + base, _CH)], rows_v)

    unm_rows = _R // _WPB2
    for t in range(unm_rows // 16):
        pltpu.sync_copy(
            zbuf_v, image_sh.at[pl.ds(img0 + k * unm_rows + t * 16, 16)])

    pltpu.sync_copy(rows_v.at[:, pl.ds(_D, _D)],
                    image_sh.at[pl.ds(img0 + _R + base, _CH), pl.ds(0, _D)])

    for t in range(_CH // 16):
        r16 = rank_v[pl.ds(t * 16, 16)]
        n16 = nidx_v[pl.ds(t * 16, 16)]
        tgt16 = jnp.where(r16 < _R, img0 + _R + n16, img0 + r16 - _R)
        tgt_v[t // 8, pl.ds((t % 8) * 16, 16)] = tgt16

    plsc.subcore_barrier()

    for j in range(_CH // 128):
        pltpu.sync_copy(rows_v.at[pl.ds(128 * j, 128)],
                        image_sh.at[tgt_v.at[j]], add=True)

    plsc.subcore_barrier()

    orow = _OUT_N // _WPB2
    pltpu.sync_copy(image_sh.at[pl.ds(img0 + k * orow, orow)],
                    out_hbm.at[pl.ds(b * _OUT_N + k * orow, orow)])


def _merge_call(xp, rankf, nidxf, h0):
    mesh = plsc.VectorSubcoreMesh(core_axis_name="c", subcore_axis_name="s")
    fn = pl.kernel(
        lambda *refs: _merge_body(h0, *refs),
        out_type=jax.ShapeDtypeStruct((_HB * _OUT_N, 2 * _D), jnp.float32),
        mesh=mesh,
        scratch_types=[
            pltpu.VMEM((_CH,), jnp.int32),
            pltpu.VMEM((_CH,), jnp.int32),
            pltpu.VMEM((_CH // 128, 128), jnp.int32),
            pltpu.VMEM((_CH, 2 * _D), jnp.float32),
            pltpu.VMEM((16, 2 * _D), jnp.float32),
            pltpu.VMEM_SHARED((_BPW * _OUT_N, 2 * _D), jnp.float32),
        ],
    )
    return fn(xp, rankf, nidxf)


def kernel(metric, x):
    mn = metric / jnp.linalg.norm(metric, axis=-1, keepdims=True)
    mp = mn.reshape(_B, _N, 2 * _D)
    xp = x.reshape(_B * _N, 2 * _D)

    outs = []
    for h in range(_B // _HB):
        sl = slice(h * _HB, (h + 1) * _HB)
        rank, nidx = _match_call(mp[sl])
        outs.append(_merge_call(xp,
                                rank.reshape(_HB * _N),
                                nidx.reshape(_HB * _N),
                                h * _HB))
    outf = jnp.concatenate([o[:, :_D] for o in outs], axis=0)
    return outf.reshape(_B, _OUT_N, _D)


# speedup vs baseline: 1.0054x; 1.0054x over previous
import jax
import jax.numpy as jnp
from jax import lax
from jax.experimental import pallas as pl
from jax.experimental.pallas import tpu as pltpu
from jax.experimental.pallas import tpu_sc as plsc

_DSQ_THRESHOLD = 2.1025002
_R = 1024

_B = 8
_N = 2048
_D = 64
_TM = 512
_NC = 2
_NS = 16
_WPB = 4
_CHUNK = _N // _WPB
_OUT_N = _R + _N


def _match_body(mp_ref, rank_ref, nidx_ref, nmr_ref, nmc_ref):
    bn = mp_ref[:, pl.ds(_D, _D)]
    bn2 = bn + bn
    bsq = bn * bn
    onesd = jnp.ones((1, _D), jnp.float32)
    b2 = lax.dot_general(onesd, bsq, (((1,), (1,)), ((), ())),
                         preferred_element_type=jnp.float32)
    ones = jnp.ones((_N, 128), jnp.float32)
    neg_inf = jnp.float32(-jnp.inf)
    for t in range(_N // _TM):
        a_t = mp_ref[pl.ds(t * _TM, _TM), pl.ds(0, _D)]
        s = lax.dot_general(a_t, bn, (((1,), (1,)), ((), ())),
                            preferred_element_type=jnp.float32)
        s2 = lax.dot_general(a_t, bn2, (((1,), (1,)), ((), ())),
                             preferred_element_type=jnp.float32)
        a2t = jnp.sum(a_t * a_t, axis=1, keepdims=True)
        dsq = a2t + b2 - s2
        sm = jnp.where(dsq > _DSQ_THRESHOLD, neg_inf, s)
        colid128 = lax.broadcasted_iota(jnp.int32, (_TM, 128), 1)
        run_v = sm[:, 0:128]
        run_i = colid128
        for cix in range(1, _N // 128):
            v = sm[:, cix * 128:(cix + 1) * 128]
            upd = v > run_v
            run_v = jnp.where(upd, v, run_v)
            run_i = jnp.where(upd, colid128 + cix * 128, run_i)
        tmax = jnp.max(run_v, axis=1, keepdims=True)
        idx = jnp.min(jnp.where(run_v == tmax, run_i, _N), axis=1)
        nidx_ref[0, pl.ds(t * _TM, _TM)] = idx
        nmr_ref[0, pl.ds(t * _TM, _TM)] = jnp.max(run_v, axis=1)
        nmc_ref[pl.ds(t * _TM, _TM), :] = tmax
    vrow = nmr_ref[...]
    for t in range(_N // _TM):
        vi = nmc_ref[pl.ds(t * _TM, _TM), :]
        colid = lax.broadcasted_iota(jnp.int32, (_TM, _N), 1)
        rowid = lax.broadcasted_iota(jnp.int32, (_TM, _N), 0) + t * _TM
        cmp = (vrow > vi) | ((vrow == vi) & (colid < rowid))
        cmpf = jnp.where(cmp, 1.0, 0.0).astype(jnp.float32)
        cnt = lax.dot_general(cmpf, ones, (((1,), (0,)), ((), ())),
                              preferred_element_type=jnp.float32)
        rank_ref[0, pl.ds(t * _TM, _TM)] = cnt[:, 0].astype(jnp.int32)


def _match_call(mp, interpret=False):
    nb = mp.shape[0]
    return pl.pallas_call(
        _match_body,
        grid=(nb,),
        in_specs=[
            pl.BlockSpec((None, _N, 2 * _D), lambda b: (b, 0, 0)),
        ],
        out_specs=[
            pl.BlockSpec((None, 1, _N), lambda b: (b, 0, 0)),
            pl.BlockSpec((None, 1, _N), lambda b: (b, 0, 0)),
        ],
        out_shape=[
            jax.ShapeDtypeStruct((nb, 1, _N), jnp.int32),
            jax.ShapeDtypeStruct((nb, 1, _N), jnp.int32),
        ],
        scratch_shapes=[
            pltpu.VMEM((1, _N), jnp.float32),
            pltpu.VMEM((_N, 1), jnp.float32),
        ],
        interpret=interpret,
    )(mp)


_HB = 2
_BPW = _HB // _NC
_WPB2 = _NS // _BPW
_CH = _N // _WPB2


def _merge_body(h0, xp_hbm, rank_hbm, nidx_hbm, out_hbm,
                rank_v, nidx_v, tgt_v, rows_v, zbuf_v, image_sh):
    c = lax.axis_index("c")
    s = lax.axis_index("s")
    lb = s // _WPB2
    k = s % _WPB2
    base = k * _CH
    img0 = lb * _OUT_N
    b = c * _BPW + lb

    zero16 = jnp.zeros((16,), jnp.float32)
    for rr in range(16):
        for cc in range(2 * _D // 16):
            zbuf_v[rr, pl.ds(cc * 16, 16)] = zero16

    pltpu.sync_copy(rank_hbm.at[pl.ds(b * _N + base, _CH)], rank_v)
    pltpu.sync_copy(nidx_hbm.at[pl.ds(b * _N + base, _CH)], nidx_v)
    pltpu.sync_copy(xp_hbm.at[pl.ds((h0 + b) * _N + base, _CH)], rows_v)

    unm_rows = _R // _WPB2
    for t in range(unm_rows // 16):
        pltpu.sync_copy(
            zbuf_v, image_sh.at[pl.ds(img0 + k * unm_rows + t * 16, 16)])

    pltpu.sync_copy(rows_v.at[:, pl.ds(_D, _D)],
                    image_sh.at[pl.ds(img0 + _R + base, _CH), pl.ds(0, _D)])

    for t in range(_CH // 16):
        r16 = rank_v[pl.ds(t * 16, 16)]
        n16 = nidx_v[pl.ds(t * 16, 16)]
        tgt16 = jnp.where(r16 < _R, img0 + _R + n16, img0 + r16 - _R)
        tgt_v[t // 8, pl.ds((t % 8) * 16, 16)] = tgt16

    plsc.subcore_barrier()

    for j in range(_CH // 128):
        pltpu.sync_copy(rows_v.at[pl.ds(128 * j, 128)],
                        image_sh.at[tgt_v.at[j]], add=True)

    plsc.subcore_barrier()

    orow = _OUT_N // _WPB2
    pltpu.sync_copy(image_sh.at[pl.ds(img0 + k * orow, orow)],
                    out_hbm.at[pl.ds(b * _OUT_N + k * orow, orow)])


def _merge_call(xp, rankf, nidxf, h0):
    mesh = plsc.VectorSubcoreMesh(core_axis_name="c", subcore_axis_name="s")
    fn = pl.kernel(
        lambda *refs: _merge_body(h0, *refs),
        out_type=jax.ShapeDtypeStruct((_HB * _OUT_N, 2 * _D), jnp.float32),
        mesh=mesh,
        scratch_types=[
            pltpu.VMEM((_CH,), jnp.int32),
            pltpu.VMEM((_CH,), jnp.int32),
            pltpu.VMEM((_CH // 128, 128), jnp.int32),
            pltpu.VMEM((_CH, 2 * _D), jnp.float32),
            pltpu.VMEM((16, 2 * _D), jnp.float32),
            pltpu.VMEM_SHARED((_BPW * _OUT_N, 2 * _D), jnp.float32),
        ],
    )
    return fn(xp, rankf, nidxf)


def kernel(metric, x):
    mn = metric / jnp.linalg.norm(metric, axis=-1, keepdims=True)
    mp = mn.reshape(_B, _N, 2 * _D)
    xp = x.reshape(_B * _N, 2 * _D)

    outs = []
    for h in range(_B // _HB):
        sl = slice(h * _HB, (h + 1) * _HB)
        rank, nidx = _match_call(mp[sl])
        outs.append(_merge_call(xp,
                                rank.reshape(_HB * _N),
                                nidx.reshape(_HB * _N),
                                h * _HB))
    outf = jnp.concatenate(outs, axis=0)
    return outf[:, :_D].reshape(_B, _OUT_N, _D)
